# baseline (speedup 1.0000x reference)
import functools

import jax
import jax.numpy as jnp
from jax import lax
from jax.experimental import pallas as pl
from jax.experimental.pallas import tpu as pltpu
from jax.experimental.pallas import tpu_sc as plsc

VOCAB = 1000000
EMBED = 64
B = 4096
L = 200

NC = 2
NS = 16
NW = NC * NS
TOTAL = B * L
ROWS_W = TOTAL // NW
CHUNK = 128
NCHUNK = ROWS_W // CHUNK
IDX_ROWS = TOTAL // CHUNK
NBUF = 4
GROUPS = EMBED // 16


def _body(idx_hbm, word_hbm, pos_hbm, out_hbm,
          idx_v, pos_v, buf0, buf1, buf2, buf3,
          sem0, sem1, sem2, sem3):
    bufs = (buf0, buf1, buf2, buf3)
    sems = (sem0, sem1, sem2, sem3)
    wid = lax.axis_index("s") * NC + lax.axis_index("c")
    pltpu.sync_copy(idx_hbm.at[pl.ds(wid * NCHUNK, NCHUNK)], idx_v)
    pltpu.sync_copy(pos_hbm.at[pl.ds(0, L)], pos_v)
    out_base = wid * ROWS_W

    def start_gather(j, b):
        pltpu.async_copy(word_hbm.at[idx_v.at[j]], bufs[b], sems[b])

    def wait_gather(j, b):
        pltpu.make_async_copy(word_hbm.at[idx_v.at[j]], bufs[b], sems[b]).wait()

    def process(j, b):
        buf = bufs[b]
        base = j * CHUNK
        wait_gather(j, b)

        def rbody(r, _):
            l = lax.rem(base + r, L)
            for u in range(GROUPS):
                sl = pl.ds(u * 16, 16)
                buf[r, sl] = buf[r, sl] + pos_v[l, sl]
            return 0

        lax.fori_loop(0, CHUNK, rbody, 0)
        pltpu.sync_copy(buf, out_hbm.at[pl.ds(out_base + base, CHUNK)])

    for b in range(NBUF):
        start_gather(b, b)

    def outer(o, _):
        j = o * NBUF
        for b in range(NBUF):
            process(j + b, b)
            start_gather(j + b + NBUF, b)
        return 0

    lax.fori_loop(0, NCHUNK // NBUF - 1, outer, 0)
    for b in range(NBUF):
        process(NCHUNK - NBUF + b, b)


_emb = functools.partial(
    pl.kernel,
    out_type=jax.ShapeDtypeStruct((TOTAL, EMBED), jnp.float32),
    mesh=plsc.VectorSubcoreMesh(core_axis_name="c", subcore_axis_name="s"),
    scratch_types=[
        pltpu.VMEM((NCHUNK, CHUNK), jnp.int32),
        pltpu.VMEM((L, EMBED), jnp.float32),
        pltpu.VMEM((CHUNK, EMBED), jnp.float32),
        pltpu.VMEM((CHUNK, EMBED), jnp.float32),
        pltpu.VMEM((CHUNK, EMBED), jnp.float32),
        pltpu.VMEM((CHUNK, EMBED), jnp.float32),
        pltpu.SemaphoreType.DMA,
        pltpu.SemaphoreType.DMA,
        pltpu.SemaphoreType.DMA,
        pltpu.SemaphoreType.DMA,
    ],
    compiler_params=pltpu.CompilerParams(use_tc_tiling_on_sc=False),
)(_body)


def kernel(inputs, word_table, pos_table):
    idx2d = inputs.reshape(IDX_ROWS, CHUNK)
    out = _emb(idx2d, word_table, pos_table)
    return out.reshape(B, L, EMBED)
